# baseline (speedup 1.0000x reference)
import functools

import jax
import jax.numpy as jnp
from jax import lax
from jax.experimental import pallas as pl
from jax.experimental.pallas import tpu as pltpu
from jax.experimental.pallas import tpu_sc as plsc

IN_F = 128
K = 20
XMIN = -3.0
XMAX = 3.0
SCALE = (K - 1) / (XMAX - XMIN)
OFFS = -XMIN * SCALE
L = 16
NC = 2
NS = 16
NW = NC * NS
NCHUNK = 2
TAB = K * IN_F

B_TC = 8192
TC_BLOCK = 1024


def _sc_body(sc_rows, row0, x_hbm, ct_hbm, out_hbm, xbuf, tab, dtab, etab,
             outbuf, *sems):
    rows_per_w = sc_rows // NW
    wid = lax.axis_index("s") * NC + lax.axis_index("c")
    base = wid * rows_per_w
    rows_per_chunk = rows_per_w // NCHUNK
    groups_per_chunk = rows_per_chunk // L

    copies = [
        pltpu.async_copy(
            x_hbm.at[pl.ds(row0 + base + i * rows_per_chunk, rows_per_chunk)],
            xbuf.at[pl.ds(i * rows_per_chunk, rows_per_chunk)],
            sems[i],
        )
        for i in range(NCHUNK)
    ]
    pltpu.sync_copy(ct_hbm, tab)

    @pl.loop(0, K - 1)
    def _mkd(k):
        kf = k.astype(jnp.float32)
        for v in range(IN_F // L):
            o = k * IN_F + v * L
            lo = tab[pl.ds(o, L)]
            hi = tab[pl.ds(o + IN_F, L)]
            d = hi - lo
            dtab[pl.ds(o, L)] = d
            etab[pl.ds(o, L)] = lo - kf * d

    lane = lax.iota(jnp.int32, L)

    for i in range(NCHUNK):
        copies[i].wait()

        @plsc.parallel_loop(i * groups_per_chunk, (i + 1) * groups_per_chunk)
        def _group(g):
            rsvec = jnp.zeros((L,), jnp.float32)
            for j in range(L):
                r = g * L + j
                acc = jnp.zeros((L,), jnp.float32)
                for v in range(IN_F // L):
                    xv = xbuf[r, pl.ds(v * L, L)]
                    t = xv * SCALE + OFFS
                    t0 = jnp.clip(t, 0.0, float(K - 2)).astype(jnp.int32)
                    idx = t0 * IN_F + (lane + v * L)
                    ee = plsc.load_gather(etab, [idx])
                    dd = plsc.load_gather(dtab, [idx])
                    acc = acc + (ee + t * dd)
                rsvec = jnp.where(lane == j, jnp.sum(acc), rsvec)
            outbuf[pl.ds(g * L, L)] = rsvec

    pltpu.sync_copy(outbuf, out_hbm.at[pl.ds(base, rows_per_w)])


def _sc_part(x, ct, row0, sc_rows):
    rows_per_w = sc_rows // NW

    def body(x_hbm, ct_hbm, out_hbm, xbuf, tab, dtab, etab, outbuf, *sems):
        _sc_body(sc_rows, row0, x_hbm, ct_hbm, out_hbm, xbuf, tab, dtab,
                 etab, outbuf, *sems)

    f = pl.kernel(
        body,
        out_type=jax.ShapeDtypeStruct((sc_rows,), jnp.float32),
        mesh=plsc.VectorSubcoreMesh(core_axis_name="c", subcore_axis_name="s"),
        compiler_params=pltpu.CompilerParams(needs_layout_passes=False),
        scratch_types=[
            pltpu.VMEM((rows_per_w, IN_F), jnp.float32),
            pltpu.VMEM((TAB,), jnp.float32),
            pltpu.VMEM((TAB,), jnp.float32),
            pltpu.VMEM((TAB,), jnp.float32),
            pltpu.VMEM((rows_per_w,), jnp.float32),
        ] + [pltpu.SemaphoreType.DMA] * NCHUNK,
    )
    return f(x, ct)


def _tc_kernel(x_ref, ct_ref, o_ref):
    nv = TC_BLOCK // 8
    t = x_ref[...].reshape(nv, 8, IN_F) * SCALE + OFFS
    d_k = ct_ref[1:2] - ct_ref[0:1]
    e_k = ct_ref[0:1]
    ee = jnp.broadcast_to(e_k, t.shape)
    dd = jnp.broadcast_to(d_k, t.shape)
    for k in range(1, K - 1):
        d_k = ct_ref[k + 1:k + 2] - ct_ref[k:k + 1]
        e_k = ct_ref[k:k + 1] - float(k) * d_k
        m = t >= float(k)
        ee = jnp.where(m, e_k, ee)
        dd = jnp.where(m, d_k, dd)
    val = ee + t * dd
    o_ref[...] = jnp.sum(val, axis=2).reshape(TC_BLOCK)


def _tc_part(x, ct8, n_rows, row0=0):
    grid = (n_rows // TC_BLOCK,)
    blk0 = row0 // TC_BLOCK
    return pl.pallas_call(
        _tc_kernel,
        grid=grid,
        in_specs=[
            pl.BlockSpec((TC_BLOCK, IN_F), lambda i: (blk0 + i, 0)),
            pl.BlockSpec((K, 8, IN_F), lambda i: (0, 0, 0)),
        ],
        out_specs=pl.BlockSpec((TC_BLOCK,), lambda i: (i,)),
        out_shape=jax.ShapeDtypeStruct((n_rows,), jnp.float32),
        compiler_params=pltpu.CompilerParams(
            dimension_semantics=("parallel",),
        ),
    )(x, ct8)


def kernel(x, coeffs):
    batch = x.shape[0]
    ct2d = coeffs.T.reshape(K, IN_F)
    ct = ct2d.reshape(TAB)
    ct8 = jnp.broadcast_to(ct2d[:, None, :], (K, 8, IN_F))
    sc_rows = batch - B_TC
    out_sc = _sc_part(x, ct, B_TC, sc_rows)
    out_tc = _tc_part(x, ct8, B_TC)
    return jnp.concatenate([out_tc, out_sc])


# speedup vs baseline: 1.0156x; 1.0156x over previous
import functools

import jax
import jax.numpy as jnp
from jax import lax
from jax.experimental import pallas as pl
from jax.experimental.pallas import tpu as pltpu
from jax.experimental.pallas import tpu_sc as plsc

IN_F = 128
K = 20
XMIN = -3.0
XMAX = 3.0
SCALE = (K - 1) / (XMAX - XMIN)
OFFS = -XMIN * SCALE
L = 16
NC = 2
NS = 16
NW = NC * NS
NCHUNK = 4
TAB = K * IN_F

B_TC = 8192
TC_BLOCK = 1024


def _sc_body(sc_rows, row0, x_hbm, ct_hbm, out_hbm, xbuf, tab, dtab, etab,
             outbuf, *sems):
    rows_per_w = sc_rows // NW
    wid = lax.axis_index("s") * NC + lax.axis_index("c")
    base = wid * rows_per_w
    rows_per_chunk = rows_per_w // NCHUNK
    groups_per_chunk = rows_per_chunk // L

    copies = [
        pltpu.async_copy(
            x_hbm.at[pl.ds(row0 + base + i * rows_per_chunk, rows_per_chunk)],
            xbuf.at[pl.ds(i * rows_per_chunk, rows_per_chunk)],
            sems[i],
        )
        for i in range(NCHUNK)
    ]
    pltpu.sync_copy(ct_hbm, tab)

    @pl.loop(0, K - 1)
    def _mkd(k):
        kf = k.astype(jnp.float32)
        for v in range(IN_F // L):
            o = k * IN_F + v * L
            lo = tab[pl.ds(o, L)]
            hi = tab[pl.ds(o + IN_F, L)]
            d = hi - lo
            dtab[pl.ds(o, L)] = d
            etab[pl.ds(o, L)] = lo - kf * d

    lane = lax.iota(jnp.int32, L)

    @pl.loop(0, rows_per_w // L)
    def _group(g):
        for i in range(NCHUNK):
            @pl.when(g == i * groups_per_chunk)
            def _wait():
                copies[i].wait()

        rsvec = jnp.zeros((L,), jnp.float32)
        for j in range(L):
            r = g * L + j
            acc = jnp.zeros((L,), jnp.float32)
            for v in range(IN_F // L):
                xv = xbuf[r, pl.ds(v * L, L)]
                t = xv * SCALE + OFFS
                t0 = jnp.clip(t, 0.0, float(K - 2)).astype(jnp.int32)
                idx = t0 * IN_F + (lane + v * L)
                ee = plsc.load_gather(etab, [idx])
                dd = plsc.load_gather(dtab, [idx])
                acc = acc + (ee + t * dd)
            rsvec = jnp.where(lane == j, jnp.sum(acc), rsvec)
        outbuf[pl.ds(g * L, L)] = rsvec

    pltpu.sync_copy(outbuf, out_hbm.at[pl.ds(base, rows_per_w)])


def _sc_part(x, ct, row0, sc_rows):
    rows_per_w = sc_rows // NW

    def body(x_hbm, ct_hbm, out_hbm, xbuf, tab, dtab, etab, outbuf, *sems):
        _sc_body(sc_rows, row0, x_hbm, ct_hbm, out_hbm, xbuf, tab, dtab,
                 etab, outbuf, *sems)

    f = pl.kernel(
        body,
        out_type=jax.ShapeDtypeStruct((sc_rows,), jnp.float32),
        mesh=plsc.VectorSubcoreMesh(core_axis_name="c", subcore_axis_name="s"),
        compiler_params=pltpu.CompilerParams(needs_layout_passes=False),
        scratch_types=[
            pltpu.VMEM((rows_per_w, IN_F), jnp.float32),
            pltpu.VMEM((TAB,), jnp.float32),
            pltpu.VMEM((TAB,), jnp.float32),
            pltpu.VMEM((TAB,), jnp.float32),
            pltpu.VMEM((rows_per_w,), jnp.float32),
        ] + [pltpu.SemaphoreType.DMA] * NCHUNK,
    )
    return f(x, ct)


def _tc_kernel(x_ref, ct_ref, o_ref):
    nv = TC_BLOCK // 8
    t = x_ref[...].reshape(nv, 8, IN_F) * SCALE + OFFS
    d_k = ct_ref[1:2] - ct_ref[0:1]
    e_k = ct_ref[0:1]
    ee = jnp.broadcast_to(e_k, t.shape)
    dd = jnp.broadcast_to(d_k, t.shape)
    for k in range(1, K - 1):
        d_k = ct_ref[k + 1:k + 2] - ct_ref[k:k + 1]
        e_k = ct_ref[k:k + 1] - float(k) * d_k
        m = t >= float(k)
        ee = jnp.where(m, e_k, ee)
        dd = jnp.where(m, d_k, dd)
    val = ee + t * dd
    o_ref[...] = jnp.sum(val, axis=2).reshape(TC_BLOCK)


def _tc_part(x, ct8, n_rows, row0=0):
    grid = (n_rows // TC_BLOCK,)
    blk0 = row0 // TC_BLOCK
    return pl.pallas_call(
        _tc_kernel,
        grid=grid,
        in_specs=[
            pl.BlockSpec((TC_BLOCK, IN_F), lambda i: (blk0 + i, 0)),
            pl.BlockSpec((K, 8, IN_F), lambda i: (0, 0, 0)),
        ],
        out_specs=pl.BlockSpec((TC_BLOCK,), lambda i: (i,)),
        out_shape=jax.ShapeDtypeStruct((n_rows,), jnp.float32),
        compiler_params=pltpu.CompilerParams(
            dimension_semantics=("parallel",),
        ),
    )(x, ct8)


def kernel(x, coeffs):
    batch = x.shape[0]
    ct2d = coeffs.T.reshape(K, IN_F)
    ct = ct2d.reshape(TAB)
    ct8 = jnp.broadcast_to(ct2d[:, None, :], (K, 8, IN_F))
    sc_rows = batch - B_TC
    out_sc = _sc_part(x, ct, B_TC, sc_rows)
    out_tc = _tc_part(x, ct8, B_TC)
    return jnp.concatenate([out_tc, out_sc])
